# baseline (speedup 1.0000x reference)
import functools

import jax
import jax.numpy as jnp
from jax import lax
from jax.experimental import pallas as pl
from jax.experimental.pallas import tpu as pltpu
from jax.experimental.pallas import tpu_sc as plsc

EPS_LN = 1e-12


def _sc_gather(table, ids_flat):
    T = ids_flat.shape[0]
    V, D = table.shape

    info = plsc.get_sparse_core_info()
    NC, NS = info.num_cores, info.num_subcores
    NW = NC * NS
    per = T // NW
    C = 64
    n_chunks = per // C

    mesh = plsc.VectorSubcoreMesh(core_axis_name="c", subcore_axis_name="s")

    @functools.partial(
        pl.kernel,
        mesh=mesh,
        out_type=jax.ShapeDtypeStruct((T, D), jnp.float32),
        scratch_types=[
            pltpu.VMEM((C,), jnp.int32),
            pltpu.VMEM((C,), jnp.int32),
            pltpu.VMEM((C, D), jnp.float32),
            pltpu.VMEM((C, D), jnp.float32),
            pltpu.SemaphoreType.DMA,
            pltpu.SemaphoreType.DMA,
        ],
    )
    def gather_kernel(table_hbm, ids_hbm, out_hbm, idx0, idx1, rows0, rows1,
                      sem0, sem1):
        cid = lax.axis_index("c")
        sid = lax.axis_index("s")
        wid = sid * NC + cid
        base = wid * per

        idx_v = (idx0, idx1)
        rows_v = (rows0, rows1)
        sems = (sem0, sem1)

        def start(i):
            b = i % 2
            off = base + i * C
            pltpu.sync_copy(ids_hbm.at[pl.ds(off, C)], idx_v[b])
            return pltpu.async_copy(table_hbm.at[idx_v[b]], rows_v[b], sems[b])

        cp = start(0)
        for i in range(n_chunks):
            nxt = start(i + 1) if i + 1 < n_chunks else None
            cp.wait()
            pltpu.sync_copy(rows_v[i % 2], out_hbm.at[pl.ds(base + i * C, C)])
            cp = nxt

    return gather_kernel(table, ids_flat)


def _tc_body(w_ref, tt_ref, pos_ref, wtt_ref, lnw_ref, lnb_ref, o_ref):
    x = w_ref[...] + pos_ref[...]
    ttf = tt_ref[0, 0, :]
    w0 = wtt_ref[0, :]
    w1 = wtt_ref[1, :]
    x = x + w0[None, :] + ttf[:, None] * (w1 - w0)[None, :]
    mu = jnp.mean(x, axis=-1, keepdims=True)
    xc = x - mu
    var = jnp.mean(xc * xc, axis=-1, keepdims=True)
    inv = lax.rsqrt(var + EPS_LN)
    o_ref[...] = xc * inv * lnw_ref[0, :][None, :] + lnb_ref[0, :][None, :]


def _tc_finish(word_rows, ttf3, W_pos, W_token_type, ln_w2, ln_b2, seq, batch):
    T, D = word_rows.shape
    BT = 256
    PB = seq // BT
    grid = (PB, batch)

    return pl.pallas_call(
        _tc_body,
        grid=grid,
        in_specs=[
            pl.BlockSpec((BT, D), lambda p, b: (b * PB + p, 0)),
            pl.BlockSpec((1, 1, BT), lambda p, b: (b * PB + p, 0, 0)),
            pl.BlockSpec((BT, D), lambda p, b: (p, 0)),
            pl.BlockSpec((2, D), lambda p, b: (0, 0)),
            pl.BlockSpec((1, D), lambda p, b: (0, 0)),
            pl.BlockSpec((1, D), lambda p, b: (0, 0)),
        ],
        out_specs=pl.BlockSpec((BT, D), lambda p, b: (b * PB + p, 0)),
        out_shape=jax.ShapeDtypeStruct((T, D), jnp.float32),
    )(word_rows, ttf3, W_pos, W_token_type, ln_w2, ln_b2)


def kernel(input_ids, token_type_ids, W_E, W_pos, W_token_type, ln_w, ln_b):
    B, S = input_ids.shape
    D = W_E.shape[1]
    T = B * S

    ids = input_ids.reshape(T).astype(jnp.int32)
    ttf3 = token_type_ids.reshape(T // 256, 1, 256).astype(jnp.float32)

    word_rows = _sc_gather(W_E, ids)
    out = _tc_finish(
        word_rows,
        ttf3,
        W_pos,
        W_token_type,
        ln_w.reshape(1, D),
        ln_b.reshape(1, D),
        S,
        B,
    )
    return out.reshape(B, S, D)


# speedup vs baseline: 1.0696x; 1.0696x over previous
import functools

import jax
import jax.numpy as jnp
from jax import lax
from jax.experimental import pallas as pl
from jax.experimental.pallas import tpu as pltpu
from jax.experimental.pallas import tpu_sc as plsc

EPS_LN = 1e-12
K_SLICES = 4
BT = 256


def _sc_gather_slice(table, ids_flat, k, B, S, K):
    V, D = table.shape
    Sk = S // K
    Tk = B * Sk

    info = plsc.get_sparse_core_info()
    NC, NS = info.num_cores, info.num_subcores
    NW = NC * NS
    wpb = NW // B
    per = Sk // wpb
    C = min(64, per)
    n_chunks = per // C

    mesh = plsc.VectorSubcoreMesh(core_axis_name="c", subcore_axis_name="s")

    @functools.partial(
        pl.kernel,
        mesh=mesh,
        out_type=jax.ShapeDtypeStruct((Tk, D), jnp.float32),
        scratch_types=[
            pltpu.VMEM((C,), jnp.int32),
            pltpu.VMEM((C,), jnp.int32),
            pltpu.VMEM((C, D), jnp.float32),
            pltpu.VMEM((C, D), jnp.float32),
            pltpu.SemaphoreType.DMA,
            pltpu.SemaphoreType.DMA,
        ],
    )
    def gather_kernel(table_hbm, ids_hbm, out_hbm, idx0, idx1, rows0, rows1,
                      sem0, sem1):
        cid = lax.axis_index("c")
        sid = lax.axis_index("s")
        wid = sid * NC + cid
        b = wid // wpb
        j = wid % wpb
        ids_base = b * S + k * Sk + j * per
        out_base = b * Sk + j * per

        idx_v = (idx0, idx1)
        rows_v = (rows0, rows1)
        sems = (sem0, sem1)

        def start(i):
            buf = i % 2
            pltpu.sync_copy(ids_hbm.at[pl.ds(ids_base + i * C, C)], idx_v[buf])
            return pltpu.async_copy(
                table_hbm.at[idx_v[buf]], rows_v[buf], sems[buf])

        cp = start(0)
        for i in range(n_chunks):
            nxt = start(i + 1) if i + 1 < n_chunks else None
            cp.wait()
            pltpu.sync_copy(rows_v[i % 2], out_hbm.at[pl.ds(out_base + i * C, C)])
            cp = nxt

    return gather_kernel(table, ids_flat)


def _tc_body(*refs):
    w_ref, tt_ref, pos_ref, wtt_ref, lnw_ref, lnb_ref, o_ref = refs[-7:]
    x = w_ref[...] + pos_ref[...]
    ttf = tt_ref[0, 0, :]
    w0 = wtt_ref[0, :]
    w1 = wtt_ref[1, :]
    x = x + w0[None, :] + ttf[:, None] * (w1 - w0)[None, :]
    mu = jnp.mean(x, axis=-1, keepdims=True)
    xc = x - mu
    var = jnp.mean(xc * xc, axis=-1, keepdims=True)
    inv = lax.rsqrt(var + EPS_LN)
    o_ref[...] = xc * inv * lnw_ref[0, :][None, :] + lnb_ref[0, :][None, :]


def _tc_finish_slice(prev_out, word_k, ttf3, W_pos, W_token_type, ln_w2,
                     ln_b2, k, B, S, K, T):
    D = word_k.shape[1]
    Sk = S // K
    PB = Sk // BT
    SB = S // BT
    grid = (PB, B)

    in_specs = [
        pl.BlockSpec((BT, D), lambda p, b: (b * PB + p, 0)),
        pl.BlockSpec((1, 1, BT), lambda p, b: (b * SB + k * PB + p, 0, 0)),
        pl.BlockSpec((BT, D), lambda p, b: (k * PB + p, 0)),
        pl.BlockSpec((2, D), lambda p, b: (0, 0)),
        pl.BlockSpec((1, D), lambda p, b: (0, 0)),
        pl.BlockSpec((1, D), lambda p, b: (0, 0)),
    ]
    args = (word_k, ttf3, W_pos, W_token_type, ln_w2, ln_b2)
    aliases = {}
    if prev_out is not None:
        in_specs = [pl.BlockSpec(memory_space=pl.ANY)] + in_specs
        args = (prev_out,) + args
        aliases = {0: 0}

    return pl.pallas_call(
        _tc_body,
        grid=grid,
        in_specs=in_specs,
        out_specs=pl.BlockSpec((BT, D), lambda p, b: (b * SB + k * PB + p, 0)),
        out_shape=jax.ShapeDtypeStruct((T, D), jnp.float32),
        input_output_aliases=aliases,
    )(*args)


def kernel(input_ids, token_type_ids, W_E, W_pos, W_token_type, ln_w, ln_b):
    B, S = input_ids.shape
    D = W_E.shape[1]
    T = B * S
    K = K_SLICES

    ids = input_ids.reshape(T).astype(jnp.int32)
    ttf3 = token_type_ids.reshape(T // BT, 1, BT).astype(jnp.float32)
    ln_w2 = ln_w.reshape(1, D)
    ln_b2 = ln_b.reshape(1, D)

    word = [_sc_gather_slice(W_E, ids, k, B, S, K) for k in range(K)]

    out = None
    for k in range(K):
        out = _tc_finish_slice(out, word[k], ttf3, W_pos, W_token_type,
                               ln_w2, ln_b2, k, B, S, K, T)
    return out.reshape(B, S, D)


# speedup vs baseline: 1.1244x; 1.0512x over previous
import functools

import jax
import jax.numpy as jnp
from jax import lax
from jax.experimental import pallas as pl
from jax.experimental.pallas import tpu as pltpu
from jax.experimental.pallas import tpu_sc as plsc

EPS_LN = 1e-12
K_SLICES = 4
BT = 512


def _sc_gather_slice(table, input_ids, k, K):
    V, D = table.shape
    B, S = input_ids.shape
    Sk = S // K
    Tk = B * Sk

    info = plsc.get_sparse_core_info()
    NC, NS = info.num_cores, info.num_subcores
    NW = NC * NS
    wpb = NW // B
    per = Sk // wpb
    C = min(64, per)
    n_chunks = per // C

    mesh = plsc.VectorSubcoreMesh(core_axis_name="c", subcore_axis_name="s")

    @functools.partial(
        pl.kernel,
        mesh=mesh,
        out_type=jax.ShapeDtypeStruct((Tk, D), jnp.float32),
        scratch_types=[
            pltpu.VMEM((C,), jnp.int32),
            pltpu.VMEM((C,), jnp.int32),
            pltpu.VMEM((C, D), jnp.float32),
            pltpu.VMEM((C, D), jnp.float32),
            pltpu.SemaphoreType.DMA,
            pltpu.SemaphoreType.DMA,
        ],
    )
    def gather_kernel(table_hbm, ids_hbm, out_hbm, idx0, idx1, rows0, rows1,
                      sem0, sem1):
        cid = lax.axis_index("c")
        sid = lax.axis_index("s")
        wid = sid * NC + cid
        b = wid // wpb
        j = wid % wpb
        col_base = k * Sk + j * per
        out_base = b * Sk + j * per

        idx_v = (idx0, idx1)
        rows_v = (rows0, rows1)
        sems = (sem0, sem1)

        def start(i):
            buf = i % 2
            pltpu.sync_copy(ids_hbm.at[b, pl.ds(col_base + i * C, C)],
                            idx_v[buf])
            return pltpu.async_copy(
                table_hbm.at[idx_v[buf]], rows_v[buf], sems[buf])

        cp = start(0)
        for i in range(n_chunks):
            nxt = start(i + 1) if i + 1 < n_chunks else None
            cp.wait()
            pltpu.sync_copy(rows_v[i % 2], out_hbm.at[pl.ds(out_base + i * C, C)])
            cp = nxt

    return gather_kernel(table, input_ids)


def _tc_body(*refs):
    w_ref, tt_ref, pos_ref, wtt_ref, lnw_ref, lnb_ref, o_ref = refs[-7:]
    x = w_ref[...] + pos_ref[...]
    ttf = tt_ref[0, 0, :].astype(jnp.float32)
    w0 = wtt_ref[0, :]
    w1 = wtt_ref[1, :]
    x = x + w0[None, :] + ttf[:, None] * (w1 - w0)[None, :]
    mu = jnp.mean(x, axis=-1, keepdims=True)
    xc = x - mu
    var = jnp.mean(xc * xc, axis=-1, keepdims=True)
    inv = lax.rsqrt(var + EPS_LN)
    o_ref[...] = xc * inv * lnw_ref[0, :][None, :] + lnb_ref[0, :][None, :]


def _tc_finish_slice(prev_out, word_k, tt3, W_pos, W_token_type, ln_w2,
                     ln_b2, k, B, S, K, T):
    D = word_k.shape[1]
    Sk = S // K
    PB = Sk // BT
    SB = S // BT
    grid = (PB, B)

    in_specs = [
        pl.BlockSpec((BT, D), lambda p, b: (b * PB + p, 0)),
        pl.BlockSpec((1, 1, BT), lambda p, b: (b * SB + k * PB + p, 0, 0)),
        pl.BlockSpec((BT, D), lambda p, b: (k * PB + p, 0)),
        pl.BlockSpec((2, D), lambda p, b: (0, 0)),
        pl.BlockSpec((1, D), lambda p, b: (0, 0)),
        pl.BlockSpec((1, D), lambda p, b: (0, 0)),
    ]
    args = (word_k, tt3, W_pos, W_token_type, ln_w2, ln_b2)
    aliases = {}
    if prev_out is not None:
        in_specs = [pl.BlockSpec(memory_space=pl.ANY)] + in_specs
        args = (prev_out,) + args
        aliases = {0: 0}

    return pl.pallas_call(
        _tc_body,
        grid=grid,
        in_specs=in_specs,
        out_specs=pl.BlockSpec((BT, D), lambda p, b: (b * SB + k * PB + p, 0)),
        out_shape=jax.ShapeDtypeStruct((T, D), jnp.float32),
        input_output_aliases=aliases,
    )(*args)


def kernel(input_ids, token_type_ids, W_E, W_pos, W_token_type, ln_w, ln_b):
    B, S = input_ids.shape
    D = W_E.shape[1]
    T = B * S
    K = K_SLICES

    ids2 = input_ids.astype(jnp.int32)
    tt3 = token_type_ids.astype(jnp.int32).reshape(T // BT, 1, BT)
    ln_w2 = ln_w.reshape(1, D)
    ln_b2 = ln_b.reshape(1, D)

    word = [_sc_gather_slice(W_E, ids2, k, K) for k in range(K)]

    out = None
    for k in range(K):
        out = _tc_finish_slice(out, word[k], tt3, W_pos, W_token_type,
                               ln_w2, ln_b2, k, B, S, K, T)
    return out.reshape(B, S, D)


# speedup vs baseline: 1.2012x; 1.0683x over previous
import functools

import jax
import jax.numpy as jnp
from jax import lax
from jax.experimental import pallas as pl
from jax.experimental.pallas import tpu as pltpu
from jax.experimental.pallas import tpu_sc as plsc

EPS_LN = 1e-12
SLICES = (1024, 512, 512)
BT = 512


def _sc_gather_slice(table, input_ids, off, Sk):
    V, D = table.shape
    B, S = input_ids.shape
    Tk = B * Sk

    info = plsc.get_sparse_core_info()
    NC, NS = info.num_cores, info.num_subcores
    NW = NC * NS
    wpb = NW // B
    per = Sk // wpb
    C = min(64, per)
    n_chunks = per // C

    mesh = plsc.VectorSubcoreMesh(core_axis_name="c", subcore_axis_name="s")

    @functools.partial(
        pl.kernel,
        mesh=mesh,
        out_type=jax.ShapeDtypeStruct((Tk, D), jnp.float32),
        scratch_types=[
            pltpu.VMEM((C,), jnp.int32),
            pltpu.VMEM((C,), jnp.int32),
            pltpu.VMEM((C, D), jnp.float32),
            pltpu.VMEM((C, D), jnp.float32),
            pltpu.SemaphoreType.DMA,
            pltpu.SemaphoreType.DMA,
        ],
    )
    def gather_kernel(table_hbm, ids_hbm, out_hbm, idx0, idx1, rows0, rows1,
                      sem0, sem1):
        cid = lax.axis_index("c")
        sid = lax.axis_index("s")
        wid = sid * NC + cid
        b = wid // wpb
        j = wid % wpb
        col_base = off + j * per
        out_base = b * Sk + j * per

        idx_v = (idx0, idx1)
        rows_v = (rows0, rows1)
        sems = (sem0, sem1)

        def start(i):
            buf = i % 2
            pltpu.sync_copy(ids_hbm.at[b, pl.ds(col_base + i * C, C)],
                            idx_v[buf])
            return pltpu.async_copy(
                table_hbm.at[idx_v[buf]], rows_v[buf], sems[buf])

        cp = start(0)
        for i in range(n_chunks):
            nxt = start(i + 1) if i + 1 < n_chunks else None
            cp.wait()
            pltpu.sync_copy(rows_v[i % 2], out_hbm.at[pl.ds(out_base + i * C, C)])
            cp = nxt

    return gather_kernel(table, input_ids)


def _tc_body(*refs):
    w_ref, tt_ref, pos_ref, wtt_ref, lnw_ref, lnb_ref, o_ref = refs[-7:]
    x = w_ref[...] + pos_ref[...]
    ttf = tt_ref[0, 0, :].astype(jnp.float32)
    w0 = wtt_ref[0, :]
    w1 = wtt_ref[1, :]
    x = x + w0[None, :] + ttf[:, None] * (w1 - w0)[None, :]
    mu = jnp.mean(x, axis=-1, keepdims=True)
    xc = x - mu
    var = jnp.mean(xc * xc, axis=-1, keepdims=True)
    inv = lax.rsqrt(var + EPS_LN)
    o_ref[...] = xc * inv * lnw_ref[...][None, :] + lnb_ref[...][None, :]


def _tc_finish_slice(prev_out, word_k, tt3, W_pos, W_token_type, ln_w,
                     ln_b, off, Sk, B, S, T):
    D = word_k.shape[1]
    PB = Sk // BT
    SB = S // BT
    ob = off // BT
    grid = (PB, B)

    in_specs = [
        pl.BlockSpec((BT, D), lambda p, b: (b * PB + p, 0)),
        pl.BlockSpec((1, 1, BT), lambda p, b: (b * SB + ob + p, 0, 0)),
        pl.BlockSpec((BT, D), lambda p, b: (ob + p, 0)),
        pl.BlockSpec((2, D), lambda p, b: (0, 0)),
        pl.BlockSpec((D,), lambda p, b: (0,)),
        pl.BlockSpec((D,), lambda p, b: (0,)),
    ]
    args = (word_k, tt3, W_pos, W_token_type, ln_w, ln_b)
    aliases = {}
    if prev_out is not None:
        in_specs = [pl.BlockSpec(memory_space=pl.ANY)] + in_specs
        args = (prev_out,) + args
        aliases = {0: 0}

    return pl.pallas_call(
        _tc_body,
        grid=grid,
        in_specs=in_specs,
        out_specs=pl.BlockSpec((BT, D), lambda p, b: (b * SB + ob + p, 0)),
        out_shape=jax.ShapeDtypeStruct((T, D), jnp.float32),
        input_output_aliases=aliases,
    )(*args)


def kernel(input_ids, token_type_ids, W_E, W_pos, W_token_type, ln_w, ln_b):
    B, S = input_ids.shape
    D = W_E.shape[1]
    T = B * S

    ids2 = input_ids.astype(jnp.int32)
    tt3 = token_type_ids.astype(jnp.int32).reshape(T // BT, 1, BT)

    offs = [sum(SLICES[:k]) for k in range(len(SLICES))]
    word = [_sc_gather_slice(W_E, ids2, offs[k], SLICES[k])
            for k in range(len(SLICES))]

    out = None
    for k in range(len(SLICES)):
        out = _tc_finish_slice(out, word[k], tt3, W_pos, W_token_type,
                               ln_w, ln_b, offs[k], SLICES[k], B, S, T)
    return out.reshape(B, S, D)


# speedup vs baseline: 1.2121x; 1.0091x over previous
import functools

import jax
import jax.numpy as jnp
from jax import lax
from jax.experimental import pallas as pl
from jax.experimental.pallas import tpu as pltpu
from jax.experimental.pallas import tpu_sc as plsc

EPS_LN = 1e-12
SLICES = (1024, 1024)
BT = 512


def _sc_gather_slice(table, input_ids, off, Sk):
    V, D = table.shape
    B, S = input_ids.shape
    Tk = B * Sk

    info = plsc.get_sparse_core_info()
    NC, NS = info.num_cores, info.num_subcores
    NW = NC * NS
    wpb = NW // B
    per = Sk // wpb
    C = min(64, per)
    n_chunks = per // C

    mesh = plsc.VectorSubcoreMesh(core_axis_name="c", subcore_axis_name="s")

    @functools.partial(
        pl.kernel,
        mesh=mesh,
        out_type=jax.ShapeDtypeStruct((Tk, D), jnp.float32),
        scratch_types=[
            pltpu.VMEM((C,), jnp.int32),
            pltpu.VMEM((C,), jnp.int32),
            pltpu.VMEM((C, D), jnp.float32),
            pltpu.VMEM((C, D), jnp.float32),
            pltpu.SemaphoreType.DMA,
            pltpu.SemaphoreType.DMA,
        ],
    )
    def gather_kernel(table_hbm, ids_hbm, out_hbm, idx0, idx1, rows0, rows1,
                      sem0, sem1):
        cid = lax.axis_index("c")
        sid = lax.axis_index("s")
        wid = sid * NC + cid
        b = wid // wpb
        j = wid % wpb
        col_base = off + j * per
        out_base = b * Sk + j * per

        idx_v = (idx0, idx1)
        rows_v = (rows0, rows1)
        sems = (sem0, sem1)

        def start(i):
            buf = i % 2
            pltpu.sync_copy(ids_hbm.at[b, pl.ds(col_base + i * C, C)],
                            idx_v[buf])
            return pltpu.async_copy(
                table_hbm.at[idx_v[buf]], rows_v[buf], sems[buf])

        cp = start(0)
        for i in range(n_chunks):
            nxt = start(i + 1) if i + 1 < n_chunks else None
            cp.wait()
            pltpu.sync_copy(rows_v[i % 2], out_hbm.at[pl.ds(out_base + i * C, C)])
            cp = nxt

    return gather_kernel(table, input_ids)


def _tc_body(*refs):
    w_ref, tt_ref, pos_ref, wtt_ref, lnw_ref, lnb_ref, o_ref = refs[-7:]
    x = w_ref[...] + pos_ref[...]
    ttf = tt_ref[0, 0, :].astype(jnp.float32)
    w0 = wtt_ref[0, :]
    w1 = wtt_ref[1, :]
    x = x + w0[None, :] + ttf[:, None] * (w1 - w0)[None, :]
    mu = jnp.mean(x, axis=-1, keepdims=True)
    xc = x - mu
    var = jnp.mean(xc * xc, axis=-1, keepdims=True)
    inv = lax.rsqrt(var + EPS_LN)
    o_ref[...] = xc * inv * lnw_ref[...][None, :] + lnb_ref[...][None, :]


def _tc_finish_slice(prev_out, word_k, tt3, W_pos, W_token_type, ln_w,
                     ln_b, off, Sk, B, S, T):
    D = word_k.shape[1]
    PB = Sk // BT
    SB = S // BT
    ob = off // BT
    grid = (PB, B)

    in_specs = [
        pl.BlockSpec((BT, D), lambda p, b: (b * PB + p, 0)),
        pl.BlockSpec((1, 1, BT), lambda p, b: (b * SB + ob + p, 0, 0)),
        pl.BlockSpec((BT, D), lambda p, b: (ob + p, 0)),
        pl.BlockSpec((2, D), lambda p, b: (0, 0)),
        pl.BlockSpec((D,), lambda p, b: (0,)),
        pl.BlockSpec((D,), lambda p, b: (0,)),
    ]
    args = (word_k, tt3, W_pos, W_token_type, ln_w, ln_b)
    aliases = {}
    if prev_out is not None:
        in_specs = [pl.BlockSpec(memory_space=pl.ANY)] + in_specs
        args = (prev_out,) + args
        aliases = {0: 0}

    return pl.pallas_call(
        _tc_body,
        grid=grid,
        in_specs=in_specs,
        out_specs=pl.BlockSpec((BT, D), lambda p, b: (b * SB + ob + p, 0)),
        out_shape=jax.ShapeDtypeStruct((T, D), jnp.float32),
        input_output_aliases=aliases,
    )(*args)


def kernel(input_ids, token_type_ids, W_E, W_pos, W_token_type, ln_w, ln_b):
    B, S = input_ids.shape
    D = W_E.shape[1]
    T = B * S

    ids2 = input_ids.astype(jnp.int32)
    tt3 = token_type_ids.astype(jnp.int32).reshape(T // BT, 1, BT)

    offs = [sum(SLICES[:k]) for k in range(len(SLICES))]
    word = [_sc_gather_slice(W_E, ids2, offs[k], SLICES[k])
            for k in range(len(SLICES))]

    out = None
    for k in range(len(SLICES)):
        out = _tc_finish_slice(out, word[k], tt3, W_pos, W_token_type,
                               ln_w, ln_b, offs[k], SLICES[k], B, S, T)
    return out.reshape(B, S, D)


# speedup vs baseline: 1.2190x; 1.0057x over previous
import functools

import jax
import jax.numpy as jnp
from jax import lax
from jax.experimental import pallas as pl
from jax.experimental.pallas import tpu as pltpu
from jax.experimental.pallas import tpu_sc as plsc

EPS_LN = 1e-12
SLICES = (1024, 1024)
BT = 512


def _sc_gather_slice(table, input_ids, off, Sk):
    V, D = table.shape
    B, S = input_ids.shape
    Tk = B * Sk

    info = plsc.get_sparse_core_info()
    NC, NS = info.num_cores, info.num_subcores
    NW = NC * NS
    wpb = NW // B
    per = Sk // wpb
    C = min(64, per)
    n_chunks = per // C

    mesh = plsc.VectorSubcoreMesh(core_axis_name="c", subcore_axis_name="s")

    @functools.partial(
        pl.kernel,
        mesh=mesh,
        out_type=jax.ShapeDtypeStruct((Tk, D), jnp.float32),
        scratch_types=[
            pltpu.VMEM((per,), jnp.int32),
            pltpu.VMEM((C, D), jnp.float32),
            pltpu.VMEM((C, D), jnp.float32),
            pltpu.SemaphoreType.DMA,
            pltpu.SemaphoreType.DMA,
        ],
    )
    def gather_kernel(table_hbm, ids_hbm, out_hbm, idx_all, rows0, rows1,
                      sem0, sem1):
        cid = lax.axis_index("c")
        sid = lax.axis_index("s")
        wid = sid * NC + cid
        b = wid // wpb
        j = wid % wpb
        col_base = off + j * per
        out_base = b * Sk + j * per

        rows_v = (rows0, rows1)
        sems = (sem0, sem1)

        pltpu.sync_copy(ids_hbm.at[b, pl.ds(col_base, per)], idx_all)

        def start(i):
            buf = i % 2
            return pltpu.async_copy(
                table_hbm.at[idx_all.at[pl.ds(i * C, C)]], rows_v[buf],
                sems[buf])

        cp = [None] * n_chunks
        for i in range(min(2, n_chunks)):
            cp[i] = start(i)
        for i in range(n_chunks):
            cp[i].wait()
            pltpu.sync_copy(rows_v[i % 2], out_hbm.at[pl.ds(out_base + i * C, C)])
            if i + 2 < n_chunks:
                cp[i + 2] = start(i + 2)

    return gather_kernel(table, input_ids)


def _tc_body(*refs):
    w_ref, tt_ref, pos_ref, wtt_ref, lnw_ref, lnb_ref, o_ref = refs[-7:]
    x = w_ref[...] + pos_ref[...]
    ttf = tt_ref[0, 0, :].astype(jnp.float32)
    w0 = wtt_ref[0, :]
    w1 = wtt_ref[1, :]
    x = x + w0[None, :] + ttf[:, None] * (w1 - w0)[None, :]
    mu = jnp.mean(x, axis=-1, keepdims=True)
    xc = x - mu
    var = jnp.mean(xc * xc, axis=-1, keepdims=True)
    inv = lax.rsqrt(var + EPS_LN)
    o_ref[...] = xc * inv * lnw_ref[...][None, :] + lnb_ref[...][None, :]


def _tc_finish_slice(prev_out, word_k, tt3, W_pos, W_token_type, ln_w,
                     ln_b, off, Sk, B, S, T):
    D = word_k.shape[1]
    PB = Sk // BT
    SB = S // BT
    ob = off // BT
    grid = (PB, B)

    in_specs = [
        pl.BlockSpec((BT, D), lambda p, b: (b * PB + p, 0)),
        pl.BlockSpec((1, 1, BT), lambda p, b: (b * SB + ob + p, 0, 0)),
        pl.BlockSpec((BT, D), lambda p, b: (ob + p, 0)),
        pl.BlockSpec((2, D), lambda p, b: (0, 0)),
        pl.BlockSpec((D,), lambda p, b: (0,)),
        pl.BlockSpec((D,), lambda p, b: (0,)),
    ]
    args = (word_k, tt3, W_pos, W_token_type, ln_w, ln_b)
    aliases = {}
    if prev_out is not None:
        in_specs = [pl.BlockSpec(memory_space=pl.ANY)] + in_specs
        args = (prev_out,) + args
        aliases = {0: 0}

    return pl.pallas_call(
        _tc_body,
        grid=grid,
        in_specs=in_specs,
        out_specs=pl.BlockSpec((BT, D), lambda p, b: (b * SB + ob + p, 0)),
        out_shape=jax.ShapeDtypeStruct((T, D), jnp.float32),
        input_output_aliases=aliases,
    )(*args)


def kernel(input_ids, token_type_ids, W_E, W_pos, W_token_type, ln_w, ln_b):
    B, S = input_ids.shape
    D = W_E.shape[1]
    T = B * S

    ids2 = input_ids.astype(jnp.int32)
    tt3 = token_type_ids.astype(jnp.int32).reshape(T // BT, 1, BT)

    offs = [sum(SLICES[:k]) for k in range(len(SLICES))]
    word = [_sc_gather_slice(W_E, ids2, offs[k], SLICES[k])
            for k in range(len(SLICES))]

    out = None
    for k in range(len(SLICES)):
        out = _tc_finish_slice(out, word[k], tt3, W_pos, W_token_type,
                               ln_w, ln_b, offs[k], SLICES[k], B, S, T)
    return out.reshape(B, S, D)
